# baseline (speedup 1.0000x reference)
import jax
import jax.numpy as jnp
from jax import lax
from jax.experimental import pallas as pl
from jax.experimental.pallas import tpu as pltpu

N = 2048
HID = 256
OUT = 128
F32 = jnp.float32


def _dinv(adj):
    bj = 512

    def body(adj_ref, out_ref):
        j = pl.program_id(0)
        t = adj_ref[...]
        b = jnp.where(t != 0, 1.0, 0.0)
        rid = lax.broadcasted_iota(jnp.int32, t.shape, 0)
        cid = j * bj + lax.broadcasted_iota(jnp.int32, t.shape, 1)
        d = jnp.where(rid == cid, 1.0, b)
        deg = jnp.sum(d, axis=0)
        out_ref[...] = 1.0 / jnp.sqrt(deg)

    return pl.pallas_call(
        body,
        grid=(N // bj,),
        in_specs=[pl.BlockSpec((N, bj), lambda j: (0, j))],
        out_specs=pl.BlockSpec((bj,), lambda j: (j,)),
        out_shape=jax.ShapeDtypeStruct((N,), F32),
    )(adj)


def _ptmm(adj, dinv_c, y, bias=None, act=None, bm=512, bk=512, bn=512):
    m_, nc = N, y.shape[1]
    bn = min(bn, nc)
    grid = (m_ // bm, nc // bn, N // bk)
    nk = grid[2]

    def body(adj_ref, dck_ref, dcm_ref, y_ref, *rest):
        if bias is not None:
            b_ref, o_ref, acc = rest
        else:
            o_ref, acc = rest
        mi = pl.program_id(0)
        ki = pl.program_id(2)

        @pl.when(ki == 0)
        def _():
            acc[...] = jnp.zeros_like(acc)

        t = adj_ref[...]
        b01 = jnp.where(t != 0, 1.0, 0.0)
        rid = ki * bk + lax.broadcasted_iota(jnp.int32, (bk, bm), 0)
        cid = mi * bm + lax.broadcasted_iota(jnp.int32, (bk, bm), 1)
        b01 = jnp.where(rid == cid, 1.0, b01)
        s = b01 * dck_ref[...]
        acc[...] += lax.dot_general(
            s, y_ref[...], (((0,), (0,)), ((), ())),
            preferred_element_type=F32)

        @pl.when(ki == nk - 1)
        def _():
            r = acc[...] * dcm_ref[...]
            if bias is not None:
                r = r + b_ref[...]
            if act is not None:
                r = act(r)
            o_ref[...] = r

    in_specs = [
        pl.BlockSpec((bk, bm), lambda m, n, k: (k, m)),
        pl.BlockSpec((bk, 1), lambda m, n, k: (k, 0)),
        pl.BlockSpec((bm, 1), lambda m, n, k: (m, 0)),
        pl.BlockSpec((bk, bn), lambda m, n, k: (k, n)),
    ]
    args = [adj, dinv_c, dinv_c, y]
    if bias is not None:
        in_specs.append(pl.BlockSpec((1, bn), lambda m, n, k: (0, n)))
        args.append(bias)
    return pl.pallas_call(
        body,
        grid=grid,
        in_specs=in_specs,
        out_specs=pl.BlockSpec((bm, bn), lambda m, n, k: (m, n)),
        out_shape=jax.ShapeDtypeStruct((m_, nc), F32),
        scratch_shapes=[pltpu.VMEM((bm, bn), F32)],
        compiler_params=pltpu.CompilerParams(
            dimension_semantics=("parallel", "parallel", "arbitrary")),
    )(*args)


def _mm(a, b, bias=None, act=None, bm=512, bk=512, bn=512):
    m_, k_ = a.shape
    nc = b.shape[1]
    bm, bk, bn = min(bm, m_), min(bk, k_), min(bn, nc)
    grid = (m_ // bm, nc // bn, k_ // bk)
    nk = grid[2]

    def body(a_ref, b_ref, *rest):
        if bias is not None:
            bias_ref, o_ref, acc = rest
        else:
            o_ref, acc = rest
        ki = pl.program_id(2)

        @pl.when(ki == 0)
        def _():
            acc[...] = jnp.zeros_like(acc)

        acc[...] += jnp.dot(a_ref[...], b_ref[...], preferred_element_type=F32)

        @pl.when(ki == nk - 1)
        def _():
            r = acc[...]
            if bias is not None:
                r = r + bias_ref[...]
            if act is not None:
                r = act(r)
            o_ref[...] = r

    in_specs = [
        pl.BlockSpec((bm, bk), lambda m, n, k: (m, k)),
        pl.BlockSpec((bk, bn), lambda m, n, k: (k, n)),
    ]
    args = [a, b]
    if bias is not None:
        in_specs.append(pl.BlockSpec((1, bn), lambda m, n, k: (0, n)))
        args.append(bias)
    return pl.pallas_call(
        body,
        grid=grid,
        in_specs=in_specs,
        out_specs=pl.BlockSpec((bm, bn), lambda m, n, k: (m, n)),
        out_shape=jax.ShapeDtypeStruct((m_, nc), F32),
        scratch_shapes=[pltpu.VMEM((bm, bn), F32)],
        compiler_params=pltpu.CompilerParams(
            dimension_semantics=("parallel", "parallel", "arbitrary")),
    )(*args)


def _readout_disc(mask, vsum, ecat, wd, bd):
    bmr = 256

    def body(mask_ref, vs_ref, e_ref, wd_ref, bd_ref, ret_ref, reta_ref):
        rs = jnp.sum(mask_ref[...], axis=1, keepdims=True)
        vs = vs_ref[...]
        gp = vs[:, :OUT] / rs
        gap = vs[:, OUT:] / rs
        n1 = jnp.maximum(jnp.sqrt(jnp.sum(gp * gp, axis=1, keepdims=True)), 1e-12)
        n2 = jnp.maximum(jnp.sqrt(jnp.sum(gap * gap, axis=1, keepdims=True)), 1e-12)
        g = jax.nn.sigmoid(gp / n1)
        ga = jax.nn.sigmoid(gap / n2)
        e = e_ref[...]
        w = wd_ref[...]
        u = jnp.dot(e[:, :OUT], w, preferred_element_type=F32)
        ua = jnp.dot(e[:, OUT:], w, preferred_element_type=F32)
        bdv = bd_ref[0, 0]
        ret_ref[...] = jnp.concatenate([
            jnp.sum(u * g, axis=1, keepdims=True) + bdv,
            jnp.sum(ua * g, axis=1, keepdims=True) + bdv], axis=1)
        reta_ref[...] = jnp.concatenate([
            jnp.sum(ua * ga, axis=1, keepdims=True) + bdv,
            jnp.sum(u * ga, axis=1, keepdims=True) + bdv], axis=1)

    return pl.pallas_call(
        body,
        grid=(N // bmr,),
        in_specs=[
            pl.BlockSpec((bmr, N), lambda m: (m, 0)),
            pl.BlockSpec((bmr, 2 * OUT), lambda m: (m, 0)),
            pl.BlockSpec((bmr, 2 * OUT), lambda m: (m, 0)),
            pl.BlockSpec((OUT, OUT), lambda m: (0, 0)),
            pl.BlockSpec((1, 1), lambda m: (0, 0)),
        ],
        out_specs=[
            pl.BlockSpec((bmr, 2), lambda m: (m, 0)),
            pl.BlockSpec((bmr, 2), lambda m: (m, 0)),
        ],
        out_shape=[
            jax.ShapeDtypeStruct((N, 2), F32),
            jax.ShapeDtypeStruct((N, 2), F32),
        ],
    )(mask, vsum, ecat, wd, bd)


def _attention(e1, e2, aw, u_row):
    bmr = 256

    def body(e1_ref, e2_ref, aw_ref, u_ref, alpha_ref, comb_ref):
        x1 = e1_ref[...]
        x2 = e2_ref[...]
        w = aw_ref[...]
        u = u_ref[...]
        v1 = jnp.tanh(jnp.dot(x1, w, preferred_element_type=F32))
        v2 = jnp.tanh(jnp.dot(x2, w, preferred_element_type=F32))
        s1 = jnp.sum(v1 * u, axis=1, keepdims=True) + 1e-6
        s2 = jnp.sum(v2 * u, axis=1, keepdims=True) + 1e-6
        mx = jnp.maximum(s1, s2)
        p1 = jnp.exp(s1 - mx)
        p2 = jnp.exp(s2 - mx)
        den = p1 + p2
        a1 = p1 / den
        a2 = p2 / den
        alpha_ref[...] = jnp.concatenate([a1, a2], axis=1)
        comb_ref[...] = a1 * x1 + a2 * x2

    return pl.pallas_call(
        body,
        grid=(N // bmr,),
        in_specs=[
            pl.BlockSpec((bmr, OUT), lambda m: (m, 0)),
            pl.BlockSpec((bmr, OUT), lambda m: (m, 0)),
            pl.BlockSpec((OUT, OUT), lambda m: (0, 0)),
            pl.BlockSpec((1, OUT), lambda m: (0, 0)),
        ],
        out_specs=[
            pl.BlockSpec((bmr, 2), lambda m: (m, 0)),
            pl.BlockSpec((bmr, OUT), lambda m: (m, 0)),
        ],
        out_shape=[
            jax.ShapeDtypeStruct((N, 2), F32),
            jax.ShapeDtypeStruct((N, OUT), F32),
        ],
    )(e1, e2, aw, u_row)


def _encode(adj, feat, feat_sh, neigh, w1, b1, w2, b2, wd, bd):
    dinv_c = _dinv(adj).reshape(N, 1)
    k_ = w1.shape[0]
    kp = -(-k_ // 512) * 512
    if kp != k_:
        feat = jnp.pad(feat, ((0, 0), (0, kp - k_)))
        feat_sh = jnp.pad(feat_sh, ((0, 0), (0, kp - k_)))
        w1 = jnp.pad(w1, ((0, kp - k_), (0, 0)))
    ya = _mm(feat, w1)
    yb = _mm(feat_sh, w1)
    y = jnp.concatenate([ya, yb], axis=1)
    b1c = jnp.concatenate([b1, b1]).reshape(1, 2 * HID)
    z = _ptmm(adj, dinv_c, y, bias=b1c, act=jax.nn.relu)
    p1 = _mm(z[:, :HID], w2)
    p2 = _mm(z[:, HID:], w2)
    p = jnp.concatenate([p1, p2], axis=1)
    b2c = jnp.concatenate([b2, b2]).reshape(1, 2 * OUT)
    ecat = _ptmm(adj, dinv_c, p, bias=b2c)
    vsum = _mm(neigh, ecat)
    ret, ret_a = _readout_disc(neigh, vsum, ecat, wd, bd.reshape(1, 1))
    return dinv_c, ecat, ret, ret_a


def _decode(adj, dinv_c, comb, w1, b1, w2, b2):
    c1 = _mm(comb, w1)
    h = _ptmm(adj, dinv_c, c1, bias=b1.reshape(1, -1), act=jax.nn.relu)
    g = _ptmm(adj, dinv_c, h)
    d_ = w2.shape[1]
    dp = -(-d_ // 512) * 512
    if dp != d_:
        w2 = jnp.pad(w2, ((0, 0), (0, dp - d_)))
        b2 = jnp.pad(b2, ((0, dp - d_),))
    rec = _mm(g, w2, bias=b2.reshape(1, -1))
    return rec[:, :d_]


def kernel(omics1_feat_shuffle, omics2_feat_shuffle, omics1_feat, omics2_feat,
           omics1_adj, omics2_adj, omics1_graph_neigh, omics2_graph_neigh,
           o1_enc_W1, o1_enc_b1, o1_enc_W2, o1_enc_b2, o1_disc_W, o1_disc_b,
           o2_enc_W1, o2_enc_b1, o2_enc_W2, o2_enc_b2, o2_disc_W, o2_disc_b,
           o1_dec_W1, o1_dec_b1, o1_dec_W2, o1_dec_b2,
           o2_dec_W1, o2_dec_b1, o2_dec_W2, o2_dec_b2, att_w, att_u):
    dc1, ecat1, o1_ret, o1_ret_a = _encode(
        omics1_adj, omics1_feat, omics1_feat_shuffle, omics1_graph_neigh,
        o1_enc_W1, o1_enc_b1, o1_enc_W2, o1_enc_b2, o1_disc_W, o1_disc_b)
    dc2, ecat2, o2_ret, o2_ret_a = _encode(
        omics2_adj, omics2_feat, omics2_feat_shuffle, omics2_graph_neigh,
        o2_enc_W1, o2_enc_b1, o2_enc_W2, o2_enc_b2, o2_disc_W, o2_disc_b)
    o1_emb = ecat1[:, :OUT]
    o2_emb = ecat2[:, :OUT]
    alpha, comb = _attention(o1_emb, o2_emb, att_w, att_u.reshape(1, OUT))
    o1_rec = _decode(omics1_adj, dc1, comb, o1_dec_W1, o1_dec_b1,
                     o1_dec_W2, o1_dec_b2)
    o2_rec = _decode(omics2_adj, dc2, comb, o2_dec_W1, o2_dec_b1,
                     o2_dec_W2, o2_dec_b2)
    return (o1_emb, o1_rec, o1_ret, o1_ret_a,
            o2_emb, o2_rec, o2_ret, o2_ret_a, comb, alpha)


# speedup vs baseline: 1.1987x; 1.1987x over previous
import jax
import jax.numpy as jnp
from jax import lax
from jax.experimental import pallas as pl
from jax.experimental.pallas import tpu as pltpu

N = 2048
HID = 256
OUT = 128
F32 = jnp.float32
BF16 = jnp.bfloat16


def _dinv(adj):
    bj = 512

    def body(adj_ref, out_ref):
        j = pl.program_id(0)
        t = adj_ref[...]
        b = jnp.where(t != 0, 1.0, 0.0)
        rid = lax.broadcasted_iota(jnp.int32, t.shape, 0)
        cid = j * bj + lax.broadcasted_iota(jnp.int32, t.shape, 1)
        d = jnp.where(rid == cid, 1.0, b)
        deg = jnp.sum(d, axis=0)
        out_ref[...] = 1.0 / jnp.sqrt(deg)

    return pl.pallas_call(
        body,
        grid=(N // bj,),
        in_specs=[pl.BlockSpec((N, bj), lambda j: (0, j))],
        out_specs=pl.BlockSpec((bj,), lambda j: (j,)),
        out_shape=jax.ShapeDtypeStruct((N,), F32),
    )(adj)


def _adj_tile_bf16(adj_ref, ki, mi, bk, bm):
    t = adj_ref[...]
    b01 = jnp.where(t != 0, 1.0, 0.0)
    rid = ki * bk + lax.broadcasted_iota(jnp.int32, (bk, bm), 0)
    cid = mi * bm + lax.broadcasted_iota(jnp.int32, (bk, bm), 1)
    return jnp.where(rid == cid, 1.0, b01).astype(BF16)


_TN = (((0,), (0,)), ((), ()))
_NN = (((1,), (0,)), ((), ()))


def _ptmm2(adj, dinv_c, y1, y2, bias=None, act=None, bm=512, bk=512, bn=512):
    nc = y1.shape[1]
    bn = min(bn, nc)
    grid = (N // bm, nc // bn, N // bk)
    nk = grid[2]

    def body(adj_ref, dck_ref, dcm_ref, y1_ref, y2_ref, *rest):
        if bias is not None:
            b_ref = rest[0]
            rest = rest[1:]
        o1_ref, o2_ref, acc1, acc2 = rest
        mi = pl.program_id(0)
        ki = pl.program_id(2)

        @pl.when(ki == 0)
        def _():
            acc1[...] = jnp.zeros_like(acc1)
            acc2[...] = jnp.zeros_like(acc2)

        s = _adj_tile_bf16(adj_ref, ki, mi, bk, bm)
        dk = dck_ref[...]
        ya = (y1_ref[...] * dk).astype(BF16)
        yb = (y2_ref[...] * dk).astype(BF16)
        acc1[...] += lax.dot_general(s, ya, _TN, preferred_element_type=F32)
        acc2[...] += lax.dot_general(s, yb, _TN, preferred_element_type=F32)

        @pl.when(ki == nk - 1)
        def _():
            dm = dcm_ref[...]
            r1 = acc1[...] * dm
            r2 = acc2[...] * dm
            if bias is not None:
                r1 = r1 + b_ref[...]
                r2 = r2 + b_ref[...]
            if act is not None:
                r1 = act(r1)
                r2 = act(r2)
            o1_ref[...] = r1
            o2_ref[...] = r2

    in_specs = [
        pl.BlockSpec((bk, bm), lambda m, n, k: (k, m)),
        pl.BlockSpec((bk, 1), lambda m, n, k: (k, 0)),
        pl.BlockSpec((bm, 1), lambda m, n, k: (m, 0)),
        pl.BlockSpec((bk, bn), lambda m, n, k: (k, n)),
        pl.BlockSpec((bk, bn), lambda m, n, k: (k, n)),
    ]
    args = [adj, dinv_c, dinv_c, y1, y2]
    if bias is not None:
        in_specs.append(pl.BlockSpec((1, bn), lambda m, n, k: (0, n)))
        args.append(bias)
    return pl.pallas_call(
        body,
        grid=grid,
        in_specs=in_specs,
        out_specs=[pl.BlockSpec((bm, bn), lambda m, n, k: (m, n))] * 2,
        out_shape=[jax.ShapeDtypeStruct((N, nc), F32)] * 2,
        scratch_shapes=[pltpu.VMEM((bm, bn), F32)] * 2,
        compiler_params=pltpu.CompilerParams(
            dimension_semantics=("parallel", "parallel", "arbitrary")),
    )(*args)


def _ptmm(adj, dinv_c, y, bias=None, act=None, bm=512, bk=512, bn=512):
    nc = y.shape[1]
    bn = min(bn, nc)
    grid = (N // bm, nc // bn, N // bk)
    nk = grid[2]

    def body(adj_ref, dck_ref, dcm_ref, y_ref, *rest):
        if bias is not None:
            b_ref = rest[0]
            rest = rest[1:]
        o_ref, acc = rest
        mi = pl.program_id(0)
        ki = pl.program_id(2)

        @pl.when(ki == 0)
        def _():
            acc[...] = jnp.zeros_like(acc)

        s = _adj_tile_bf16(adj_ref, ki, mi, bk, bm)
        ya = (y_ref[...] * dck_ref[...]).astype(BF16)
        acc[...] += lax.dot_general(s, ya, _TN, preferred_element_type=F32)

        @pl.when(ki == nk - 1)
        def _():
            r = acc[...] * dcm_ref[...]
            if bias is not None:
                r = r + b_ref[...]
            if act is not None:
                r = act(r)
            o_ref[...] = r

    in_specs = [
        pl.BlockSpec((bk, bm), lambda m, n, k: (k, m)),
        pl.BlockSpec((bk, 1), lambda m, n, k: (k, 0)),
        pl.BlockSpec((bm, 1), lambda m, n, k: (m, 0)),
        pl.BlockSpec((bk, bn), lambda m, n, k: (k, n)),
    ]
    args = [adj, dinv_c, dinv_c, y]
    if bias is not None:
        in_specs.append(pl.BlockSpec((1, bn), lambda m, n, k: (0, n)))
        args.append(bias)
    return pl.pallas_call(
        body,
        grid=grid,
        in_specs=in_specs,
        out_specs=pl.BlockSpec((bm, bn), lambda m, n, k: (m, n)),
        out_shape=jax.ShapeDtypeStruct((N, nc), F32),
        scratch_shapes=[pltpu.VMEM((bm, bn), F32)],
        compiler_params=pltpu.CompilerParams(
            dimension_semantics=("parallel", "parallel", "arbitrary")),
    )(*args)


def _mm2l(a1, a2, b, bm=512, bk=512, bn=512):
    m_, k_ = a1.shape
    nc = b.shape[1]
    bm, bn = min(bm, m_), min(bn, nc)
    bk = min(bk, -(-k_ // 256) * 256)
    nk = -(-k_ // bk)
    ragged = (k_ % bk) != 0
    grid = (m_ // bm, nc // bn, nk)

    def body(a1_ref, a2_ref, b_ref, o1_ref, o2_ref, acc1, acc2):
        ki = pl.program_id(2)

        @pl.when(ki == 0)
        def _():
            acc1[...] = jnp.zeros_like(acc1)
            acc2[...] = jnp.zeros_like(acc2)

        t1 = a1_ref[...]
        t2 = a2_ref[...]
        tb = b_ref[...]
        if ragged:
            kids = ki * bk + lax.broadcasted_iota(jnp.int32, (bm, bk), 1)
            keep = kids < k_
            t1 = jnp.where(keep, t1, 0.0)
            t2 = jnp.where(keep, t2, 0.0)
            kidb = ki * bk + lax.broadcasted_iota(jnp.int32, (bk, bn), 0)
            tb = jnp.where(kidb < k_, tb, 0.0)
        t1 = t1.astype(BF16)
        t2 = t2.astype(BF16)
        tb = tb.astype(BF16)
        acc1[...] += lax.dot_general(t1, tb, _NN, preferred_element_type=F32)
        acc2[...] += lax.dot_general(t2, tb, _NN, preferred_element_type=F32)

        @pl.when(ki == nk - 1)
        def _():
            o1_ref[...] = acc1[...]
            o2_ref[...] = acc2[...]

    return pl.pallas_call(
        body,
        grid=grid,
        in_specs=[
            pl.BlockSpec((bm, bk), lambda m, n, k: (m, k)),
            pl.BlockSpec((bm, bk), lambda m, n, k: (m, k)),
            pl.BlockSpec((bk, bn), lambda m, n, k: (k, n)),
        ],
        out_specs=[pl.BlockSpec((bm, bn), lambda m, n, k: (m, n))] * 2,
        out_shape=[jax.ShapeDtypeStruct((m_, nc), F32)] * 2,
        scratch_shapes=[pltpu.VMEM((bm, bn), F32)] * 2,
        compiler_params=pltpu.CompilerParams(
            dimension_semantics=("parallel", "parallel", "arbitrary")),
    )(a1, a2, b)


def _mm2r(a, b1, b2, bm=512, bk=512, bn=512):
    m_, k_ = a.shape
    nc = b1.shape[1]
    bm, bk, bn = min(bm, m_), min(bk, k_), min(bn, nc)
    grid = (m_ // bm, nc // bn, k_ // bk)
    nk = grid[2]

    def body(a_ref, b1_ref, b2_ref, o1_ref, o2_ref, acc1, acc2):
        ki = pl.program_id(2)

        @pl.when(ki == 0)
        def _():
            acc1[...] = jnp.zeros_like(acc1)
            acc2[...] = jnp.zeros_like(acc2)

        ta = a_ref[...].astype(BF16)
        acc1[...] += lax.dot_general(ta, b1_ref[...].astype(BF16), _NN,
                                     preferred_element_type=F32)
        acc2[...] += lax.dot_general(ta, b2_ref[...].astype(BF16), _NN,
                                     preferred_element_type=F32)

        @pl.when(ki == nk - 1)
        def _():
            o1_ref[...] = acc1[...]
            o2_ref[...] = acc2[...]

    return pl.pallas_call(
        body,
        grid=grid,
        in_specs=[
            pl.BlockSpec((bm, bk), lambda m, n, k: (m, k)),
            pl.BlockSpec((bk, bn), lambda m, n, k: (k, n)),
            pl.BlockSpec((bk, bn), lambda m, n, k: (k, n)),
        ],
        out_specs=[pl.BlockSpec((bm, bn), lambda m, n, k: (m, n))] * 2,
        out_shape=[jax.ShapeDtypeStruct((m_, nc), F32)] * 2,
        scratch_shapes=[pltpu.VMEM((bm, bn), F32)] * 2,
        compiler_params=pltpu.CompilerParams(
            dimension_semantics=("parallel", "parallel", "arbitrary")),
    )(a, b1, b2)


def _mm(a, b, bias=None, act=None, bm=512, bk=512, bn=512):
    m_, k_ = a.shape
    nc = b.shape[1]
    bm, bk = min(bm, m_), min(bk, k_)
    bn = min(bn, -(-nc // 128) * 128)
    grid = (m_ // bm, -(-nc // bn), k_ // bk)
    nk = grid[2]

    def body(a_ref, b_ref, *rest):
        if bias is not None:
            bias_ref = rest[0]
            rest = rest[1:]
        o_ref, acc = rest
        ki = pl.program_id(2)

        @pl.when(ki == 0)
        def _():
            acc[...] = jnp.zeros_like(acc)

        acc[...] += lax.dot_general(
            a_ref[...].astype(BF16), b_ref[...].astype(BF16), _NN,
            preferred_element_type=F32)

        @pl.when(ki == nk - 1)
        def _():
            r = acc[...]
            if bias is not None:
                r = r + bias_ref[...]
            if act is not None:
                r = act(r)
            o_ref[...] = r

    in_specs = [
        pl.BlockSpec((bm, bk), lambda m, n, k: (m, k)),
        pl.BlockSpec((bk, bn), lambda m, n, k: (k, n)),
    ]
    args = [a, b]
    if bias is not None:
        in_specs.append(pl.BlockSpec((1, bn), lambda m, n, k: (0, n)))
        args.append(bias)
    return pl.pallas_call(
        body,
        grid=grid,
        in_specs=in_specs,
        out_specs=pl.BlockSpec((bm, bn), lambda m, n, k: (m, n)),
        out_shape=jax.ShapeDtypeStruct((m_, nc), F32),
        scratch_shapes=[pltpu.VMEM((bm, bn), F32)],
        compiler_params=pltpu.CompilerParams(
            dimension_semantics=("parallel", "parallel", "arbitrary")),
    )(*args)


def _readout_disc(mask, vs1, vs2, e1, e2, wd, bd):
    bmr = 256

    def body(mask_ref, vs1_ref, vs2_ref, e1_ref, e2_ref, wd_ref, bd_ref,
             ret_ref, reta_ref):
        rs = jnp.sum(mask_ref[...], axis=1, keepdims=True)
        gp = vs1_ref[...] / rs
        gap = vs2_ref[...] / rs
        n1 = jnp.maximum(jnp.sqrt(jnp.sum(gp * gp, axis=1, keepdims=True)), 1e-12)
        n2 = jnp.maximum(jnp.sqrt(jnp.sum(gap * gap, axis=1, keepdims=True)), 1e-12)
        g = jax.nn.sigmoid(gp / n1)
        ga = jax.nn.sigmoid(gap / n2)
        w = wd_ref[...]
        u = jnp.dot(e1_ref[...], w, preferred_element_type=F32)
        ua = jnp.dot(e2_ref[...], w, preferred_element_type=F32)
        bdv = bd_ref[0, 0]
        ret_ref[...] = jnp.concatenate([
            jnp.sum(u * g, axis=1, keepdims=True) + bdv,
            jnp.sum(ua * g, axis=1, keepdims=True) + bdv], axis=1)
        reta_ref[...] = jnp.concatenate([
            jnp.sum(ua * ga, axis=1, keepdims=True) + bdv,
            jnp.sum(u * ga, axis=1, keepdims=True) + bdv], axis=1)

    return pl.pallas_call(
        body,
        grid=(N // bmr,),
        in_specs=[
            pl.BlockSpec((bmr, N), lambda m: (m, 0)),
            pl.BlockSpec((bmr, OUT), lambda m: (m, 0)),
            pl.BlockSpec((bmr, OUT), lambda m: (m, 0)),
            pl.BlockSpec((bmr, OUT), lambda m: (m, 0)),
            pl.BlockSpec((bmr, OUT), lambda m: (m, 0)),
            pl.BlockSpec((OUT, OUT), lambda m: (0, 0)),
            pl.BlockSpec((1, 1), lambda m: (0, 0)),
        ],
        out_specs=[
            pl.BlockSpec((bmr, 2), lambda m: (m, 0)),
            pl.BlockSpec((bmr, 2), lambda m: (m, 0)),
        ],
        out_shape=[
            jax.ShapeDtypeStruct((N, 2), F32),
            jax.ShapeDtypeStruct((N, 2), F32),
        ],
    )(mask, vs1, vs2, e1, e2, wd, bd)


def _attention(e1, e2, aw, u_row):
    bmr = 256

    def body(e1_ref, e2_ref, aw_ref, u_ref, alpha_ref, comb_ref):
        x1 = e1_ref[...]
        x2 = e2_ref[...]
        w = aw_ref[...]
        u = u_ref[...]
        v1 = jnp.tanh(jnp.dot(x1, w, preferred_element_type=F32))
        v2 = jnp.tanh(jnp.dot(x2, w, preferred_element_type=F32))
        s1 = jnp.sum(v1 * u, axis=1, keepdims=True) + 1e-6
        s2 = jnp.sum(v2 * u, axis=1, keepdims=True) + 1e-6
        mx = jnp.maximum(s1, s2)
        p1 = jnp.exp(s1 - mx)
        p2 = jnp.exp(s2 - mx)
        den = p1 + p2
        a1 = p1 / den
        a2 = p2 / den
        alpha_ref[...] = jnp.concatenate([a1, a2], axis=1)
        comb_ref[...] = a1 * x1 + a2 * x2

    return pl.pallas_call(
        body,
        grid=(N // bmr,),
        in_specs=[
            pl.BlockSpec((bmr, OUT), lambda m: (m, 0)),
            pl.BlockSpec((bmr, OUT), lambda m: (m, 0)),
            pl.BlockSpec((OUT, OUT), lambda m: (0, 0)),
            pl.BlockSpec((1, OUT), lambda m: (0, 0)),
        ],
        out_specs=[
            pl.BlockSpec((bmr, 2), lambda m: (m, 0)),
            pl.BlockSpec((bmr, OUT), lambda m: (m, 0)),
        ],
        out_shape=[
            jax.ShapeDtypeStruct((N, 2), F32),
            jax.ShapeDtypeStruct((N, OUT), F32),
        ],
    )(e1, e2, aw, u_row)


def _encode(adj, feat, feat_sh, neigh, w1, b1, w2, b2, wd, bd):
    dinv_c = _dinv(adj).reshape(N, 1)
    ya, yb = _mm2l(feat, feat_sh, w1)
    z1, z2 = _ptmm2(adj, dinv_c, ya, yb, bias=b1.reshape(1, HID),
                    act=jax.nn.relu)
    p1, p2 = _mm2l(z1, z2, w2)
    e1, e2 = _ptmm2(adj, dinv_c, p1, p2, bias=b2.reshape(1, OUT))
    vs1, vs2 = _mm2r(neigh, e1, e2)
    ret, ret_a = _readout_disc(neigh, vs1, vs2, e1, e2, wd, bd.reshape(1, 1))
    return dinv_c, e1, ret, ret_a


def _decode(adj, dinv_c, comb, w1, b1, w2, b2):
    c1 = _mm(comb, w1)
    h = _ptmm(adj, dinv_c, c1, bias=b1.reshape(1, -1), act=jax.nn.relu)
    g = _ptmm(adj, dinv_c, h)
    return _mm(g, w2, bias=b2.reshape(1, -1))


def kernel(omics1_feat_shuffle, omics2_feat_shuffle, omics1_feat, omics2_feat,
           omics1_adj, omics2_adj, omics1_graph_neigh, omics2_graph_neigh,
           o1_enc_W1, o1_enc_b1, o1_enc_W2, o1_enc_b2, o1_disc_W, o1_disc_b,
           o2_enc_W1, o2_enc_b1, o2_enc_W2, o2_enc_b2, o2_disc_W, o2_disc_b,
           o1_dec_W1, o1_dec_b1, o1_dec_W2, o1_dec_b2,
           o2_dec_W1, o2_dec_b1, o2_dec_W2, o2_dec_b2, att_w, att_u):
    dc1, o1_emb, o1_ret, o1_ret_a = _encode(
        omics1_adj, omics1_feat, omics1_feat_shuffle, omics1_graph_neigh,
        o1_enc_W1, o1_enc_b1, o1_enc_W2, o1_enc_b2, o1_disc_W, o1_disc_b)
    dc2, o2_emb, o2_ret, o2_ret_a = _encode(
        omics2_adj, omics2_feat, omics2_feat_shuffle, omics2_graph_neigh,
        o2_enc_W1, o2_enc_b1, o2_enc_W2, o2_enc_b2, o2_disc_W, o2_disc_b)
    alpha, comb = _attention(o1_emb, o2_emb, att_w, att_u.reshape(1, OUT))
    o1_rec = _decode(omics1_adj, dc1, comb, o1_dec_W1, o1_dec_b1,
                     o1_dec_W2, o1_dec_b2)
    o2_rec = _decode(omics2_adj, dc2, comb, o2_dec_W1, o2_dec_b1,
                     o2_dec_W2, o2_dec_b2)
    return (o1_emb, o1_rec, o1_ret, o1_ret_a,
            o2_emb, o2_rec, o2_ret, o2_ret_a, comb, alpha)


# speedup vs baseline: 1.7911x; 1.4942x over previous
import jax
import jax.numpy as jnp
from jax import lax
from jax.experimental import pallas as pl
from jax.experimental.pallas import tpu as pltpu

N = 2048
HID = 256
OUT = 128
F32 = jnp.float32
BF16 = jnp.bfloat16

_TN = (((0,), (0,)), ((), ()))
_NN = (((1,), (0,)), ((), ()))
_PARAMS = pltpu.CompilerParams(
    dimension_semantics=("parallel", "parallel"))


def _prep(adj):
    bj = 512

    def body(adj_ref, b_ref, dinv_ref):
        j = pl.program_id(0)
        t = adj_ref[...]
        b = jnp.where(t != 0, 1.0, 0.0)
        rid = lax.broadcasted_iota(jnp.int32, t.shape, 0)
        cid = j * bj + lax.broadcasted_iota(jnp.int32, t.shape, 1)
        d = jnp.where(rid == cid, 1.0, b)
        b_ref[...] = d.astype(BF16)
        deg = jnp.sum(d, axis=0)
        dinv_ref[...] = 1.0 / jnp.sqrt(deg)

    return pl.pallas_call(
        body,
        grid=(N // bj,),
        in_specs=[pl.BlockSpec((N, bj), lambda j: (0, j))],
        out_specs=[
            pl.BlockSpec((N, bj), lambda j: (0, j)),
            pl.BlockSpec((bj,), lambda j: (j,)),
        ],
        out_shape=[
            jax.ShapeDtypeStruct((N, N), BF16),
            jax.ShapeDtypeStruct((N,), F32),
        ],
    )(adj)


def _ptmm(b, dinv_c, ys, bias=None, act=None, post_dinv=False,
          out_dtype=F32, bm=512, bn=512):
    nd = len(ys)
    nc = ys[0].shape[1]
    bn = min(bn, nc)
    grid = (N // bm, nc // bn)

    def body(b_ref, dcm_ref, *rest):
        y_refs = rest[:nd]
        rest = rest[nd:]
        if bias is not None:
            bias_ref = rest[0]
            rest = rest[1:]
        o_refs = rest
        s = b_ref[...]
        dm = dcm_ref[...]
        for yr, orf in zip(y_refs, o_refs):
            r = lax.dot_general(s, yr[...], _TN,
                                preferred_element_type=F32) * dm
            if bias is not None:
                r = r + bias_ref[...]
            if act is not None:
                r = act(r)
            if post_dinv:
                r = r * dm
            orf[...] = r.astype(out_dtype)

    in_specs = [
        pl.BlockSpec((N, bm), lambda m, n: (0, m)),
        pl.BlockSpec((bm, 1), lambda m, n: (m, 0)),
    ] + [pl.BlockSpec((N, bn), lambda m, n: (0, n))] * nd
    args = [b, dinv_c] + list(ys)
    if bias is not None:
        in_specs.append(pl.BlockSpec((1, bn), lambda m, n: (0, n)))
        args.append(bias)
    out = pl.pallas_call(
        body,
        grid=grid,
        in_specs=in_specs,
        out_specs=[pl.BlockSpec((bm, bn), lambda m, n: (m, n))] * nd,
        out_shape=[jax.ShapeDtypeStruct((N, nc), out_dtype)] * nd,
        compiler_params=_PARAMS,
    )(*args)
    return out


def _mm(avs, bmat, dinv_c=None, bias=None, act=None, out_dtype=F32,
        bm=512, bn=512):
    nd = len(avs)
    m_, k_ = avs[0].shape
    nc = bmat.shape[1]
    bn = min(bn, -(-nc // 128) * 128)
    grid = (m_ // bm, -(-nc // bn))
    k_main = (k_ // 128) * 128
    ragged = k_main != k_
    ktail_blk = k_main // 128

    def body(*refs):
        refs = list(refs)
        a_refs = [refs.pop(0) for _ in range(nd)]
        if ragged:
            at_refs = [refs.pop(0) for _ in range(nd)]
        b_ref = refs.pop(0)
        if ragged:
            bt_ref = refs.pop(0)
        if dinv_c is not None:
            dcm_ref = refs.pop(0)
        if bias is not None:
            bias_ref = refs.pop(0)
        o_refs = refs
        bmain = b_ref[...].astype(BF16)
        if ragged:
            kid = lax.broadcasted_iota(jnp.int32, (128, bn), 0)
            btail = jnp.where(kid < (k_ - k_main), bt_ref[...], 0.0)
            btail = btail.astype(BF16)
        for i in range(nd):
            r = lax.dot_general(a_refs[i][...].astype(BF16), bmain, _NN,
                                preferred_element_type=F32)
            if ragged:
                kida = lax.broadcasted_iota(jnp.int32, (bm, 128), 1)
                atail = jnp.where(kida < (k_ - k_main), at_refs[i][...], 0.0)
                r = r + lax.dot_general(atail.astype(BF16), btail, _NN,
                                        preferred_element_type=F32)
            if dinv_c is not None:
                r = r * dcm_ref[...]
            if bias is not None:
                r = r + bias_ref[...]
            if act is not None:
                r = act(r)
            o_refs[i][...] = r.astype(out_dtype)

    in_specs = [pl.BlockSpec((bm, k_main), lambda m, n: (m, 0))] * nd
    args = list(avs)
    if ragged:
        in_specs += [pl.BlockSpec((bm, 128),
                                  lambda m, n: (m, ktail_blk))] * nd
        args += list(avs)
    in_specs.append(pl.BlockSpec((k_main, bn), lambda m, n: (0, n)))
    args.append(bmat)
    if ragged:
        in_specs.append(pl.BlockSpec((128, bn), lambda m, n: (ktail_blk, n)))
        args.append(bmat)
    if dinv_c is not None:
        in_specs.append(pl.BlockSpec((bm, 1), lambda m, n: (m, 0)))
        args.append(dinv_c)
    if bias is not None:
        in_specs.append(pl.BlockSpec((1, bn), lambda m, n: (0, n)))
        args.append(bias)
    out = pl.pallas_call(
        body,
        grid=grid,
        in_specs=in_specs,
        out_specs=[pl.BlockSpec((bm, bn), lambda m, n: (m, n))] * nd,
        out_shape=[jax.ShapeDtypeStruct((m_, nc), out_dtype)] * nd,
        compiler_params=_PARAMS,
    )(*args)
    return out


def _vsum_readout_disc(mask, e1, e2, wd, bd):
    bmr = 512

    def body(mask_ref, e1f_ref, e2f_ref, e1r_ref, e2r_ref, wd_ref, bd_ref,
             ret_ref, reta_ref):
        mk = mask_ref[...]
        rs = jnp.sum(mk, axis=1, keepdims=True)
        mb = mk.astype(BF16)
        vs1 = lax.dot_general(mb, e1f_ref[...].astype(BF16), _NN,
                              preferred_element_type=F32)
        vs2 = lax.dot_general(mb, e2f_ref[...].astype(BF16), _NN,
                              preferred_element_type=F32)
        gp = vs1 / rs
        gap = vs2 / rs
        n1 = jnp.maximum(jnp.sqrt(jnp.sum(gp * gp, axis=1, keepdims=True)), 1e-12)
        n2 = jnp.maximum(jnp.sqrt(jnp.sum(gap * gap, axis=1, keepdims=True)), 1e-12)
        g = jax.nn.sigmoid(gp / n1)
        ga = jax.nn.sigmoid(gap / n2)
        w = wd_ref[...]
        u = jnp.dot(e1r_ref[...], w, preferred_element_type=F32)
        ua = jnp.dot(e2r_ref[...], w, preferred_element_type=F32)
        bdv = bd_ref[0, 0]
        ret_ref[...] = jnp.concatenate([
            jnp.sum(u * g, axis=1, keepdims=True) + bdv,
            jnp.sum(ua * g, axis=1, keepdims=True) + bdv], axis=1)
        reta_ref[...] = jnp.concatenate([
            jnp.sum(ua * ga, axis=1, keepdims=True) + bdv,
            jnp.sum(u * ga, axis=1, keepdims=True) + bdv], axis=1)

    return pl.pallas_call(
        body,
        grid=(N // bmr,),
        in_specs=[
            pl.BlockSpec((bmr, N), lambda m: (m, 0)),
            pl.BlockSpec((N, OUT), lambda m: (0, 0)),
            pl.BlockSpec((N, OUT), lambda m: (0, 0)),
            pl.BlockSpec((bmr, OUT), lambda m: (m, 0)),
            pl.BlockSpec((bmr, OUT), lambda m: (m, 0)),
            pl.BlockSpec((OUT, OUT), lambda m: (0, 0)),
            pl.BlockSpec((1, 1), lambda m: (0, 0)),
        ],
        out_specs=[
            pl.BlockSpec((bmr, 2), lambda m: (m, 0)),
            pl.BlockSpec((bmr, 2), lambda m: (m, 0)),
        ],
        out_shape=[
            jax.ShapeDtypeStruct((N, 2), F32),
            jax.ShapeDtypeStruct((N, 2), F32),
        ],
    )(mask, e1, e2, e1, e2, wd, bd)


def _attention(e1, e2, aw, u_row):
    bmr = 256

    def body(e1_ref, e2_ref, aw_ref, u_ref, alpha_ref, comb_ref):
        x1 = e1_ref[...]
        x2 = e2_ref[...]
        w = aw_ref[...]
        u = u_ref[...]
        v1 = jnp.tanh(jnp.dot(x1, w, preferred_element_type=F32))
        v2 = jnp.tanh(jnp.dot(x2, w, preferred_element_type=F32))
        s1 = jnp.sum(v1 * u, axis=1, keepdims=True) + 1e-6
        s2 = jnp.sum(v2 * u, axis=1, keepdims=True) + 1e-6
        mx = jnp.maximum(s1, s2)
        p1 = jnp.exp(s1 - mx)
        p2 = jnp.exp(s2 - mx)
        den = p1 + p2
        a1 = p1 / den
        a2 = p2 / den
        alpha_ref[...] = jnp.concatenate([a1, a2], axis=1)
        comb_ref[...] = a1 * x1 + a2 * x2

    return pl.pallas_call(
        body,
        grid=(N // bmr,),
        in_specs=[
            pl.BlockSpec((bmr, OUT), lambda m: (m, 0)),
            pl.BlockSpec((bmr, OUT), lambda m: (m, 0)),
            pl.BlockSpec((OUT, OUT), lambda m: (0, 0)),
            pl.BlockSpec((1, OUT), lambda m: (0, 0)),
        ],
        out_specs=[
            pl.BlockSpec((bmr, 2), lambda m: (m, 0)),
            pl.BlockSpec((bmr, OUT), lambda m: (m, 0)),
        ],
        out_shape=[
            jax.ShapeDtypeStruct((N, 2), F32),
            jax.ShapeDtypeStruct((N, OUT), F32),
        ],
    )(e1, e2, aw, u_row)


def _encode(adj, feat, feat_sh, neigh, w1, b1, w2, b2, wd, bd):
    bmat, dinv = _prep(adj)
    dinv_c = dinv.reshape(N, 1)
    ya, yb = _mm([feat, feat_sh], w1, dinv_c=dinv_c, out_dtype=BF16)
    z1, z2 = _ptmm(bmat, dinv_c, [ya, yb], bias=b1.reshape(1, HID),
                   act=jax.nn.relu, out_dtype=BF16)
    p1, p2 = _mm([z1, z2], w2, dinv_c=dinv_c, out_dtype=BF16)
    e1, e2 = _ptmm(bmat, dinv_c, [p1, p2], bias=b2.reshape(1, OUT))
    ret, ret_a = _vsum_readout_disc(neigh, e1, e2, wd, bd.reshape(1, 1))
    return bmat, dinv_c, e1, ret, ret_a


def _decode(bmat, dinv_c, comb, w1, b1, w2, b2):
    (c1,) = _mm([comb], w1, dinv_c=dinv_c, out_dtype=BF16)
    (h,) = _ptmm(bmat, dinv_c, [c1], bias=b1.reshape(1, -1),
                 act=jax.nn.relu, post_dinv=True, out_dtype=BF16)
    (g,) = _ptmm(bmat, dinv_c, [h], out_dtype=BF16)
    (rec,) = _mm([g], w2, bias=b2.reshape(1, -1))
    return rec


def kernel(omics1_feat_shuffle, omics2_feat_shuffle, omics1_feat, omics2_feat,
           omics1_adj, omics2_adj, omics1_graph_neigh, omics2_graph_neigh,
           o1_enc_W1, o1_enc_b1, o1_enc_W2, o1_enc_b2, o1_disc_W, o1_disc_b,
           o2_enc_W1, o2_enc_b1, o2_enc_W2, o2_enc_b2, o2_disc_W, o2_disc_b,
           o1_dec_W1, o1_dec_b1, o1_dec_W2, o1_dec_b2,
           o2_dec_W1, o2_dec_b1, o2_dec_W2, o2_dec_b2, att_w, att_u):
    b1m, dc1, o1_emb, o1_ret, o1_ret_a = _encode(
        omics1_adj, omics1_feat, omics1_feat_shuffle, omics1_graph_neigh,
        o1_enc_W1, o1_enc_b1, o1_enc_W2, o1_enc_b2, o1_disc_W, o1_disc_b)
    b2m, dc2, o2_emb, o2_ret, o2_ret_a = _encode(
        omics2_adj, omics2_feat, omics2_feat_shuffle, omics2_graph_neigh,
        o2_enc_W1, o2_enc_b1, o2_enc_W2, o2_enc_b2, o2_disc_W, o2_disc_b)
    alpha, comb = _attention(o1_emb, o2_emb, att_w, att_u.reshape(1, OUT))
    o1_rec = _decode(b1m, dc1, comb, o1_dec_W1, o1_dec_b1,
                     o1_dec_W2, o1_dec_b2)
    o2_rec = _decode(b2m, dc2, comb, o2_dec_W1, o2_dec_b1,
                     o2_dec_W2, o2_dec_b2)
    return (o1_emb, o1_rec, o1_ret, o1_ret_a,
            o2_emb, o2_rec, o2_ret, o2_ret_a, comb, alpha)
